# baseline (speedup 1.0000x reference)
import functools

import jax
import jax.numpy as jnp
from jax import lax
from jax.experimental import pallas as pl
from jax.experimental.pallas import tpu as pltpu, tpu_sc as plsc

B, V, N, H, W = 2, 4, 100000, 512, 512
HW = H * W
NPAD = 100352
PCHUNK = 12544
NPC = NPAD // PCHUNK
QPIX = HW // 4
CCHUNK = 2048
NCC = QPIX // CCHUNK
PIXOFF = QPIX


def _records_body(feat_ref, op_ref, out_ref):
    ch = pl.program_id(0)
    f = feat_ref[...]
    av = jnp.clip(op_ref[..., 0], 0.0, 1.0)
    v1 = jax.nn.sigmoid(f[:, 0]) * av
    v2 = jax.nn.sigmoid(f[:, 1]) * av
    v3 = jax.nn.sigmoid(f[:, 2]) * av
    val = jnp.where(ch == 0, av,
                    jnp.where(ch == 1, v1, jnp.where(ch == 2, v2, v3)))
    out_ref[...] = lax.bitcast_convert_type(val, jnp.int32)


def _records(feat2d, op2d):
    rows = feat2d.shape[0]
    blk = 7168
    return pl.pallas_call(
        _records_body,
        grid=(4, rows // blk),
        in_specs=[
            pl.BlockSpec((blk, 16), lambda c, i: (i, 0)),
            pl.BlockSpec((blk, 1), lambda c, i: (i, 0)),
        ],
        out_specs=pl.BlockSpec((blk,), lambda c, i: (c * (rows // blk) + i,)),
        out_shape=jax.ShapeDtypeStruct((4 * rows,), jnp.int32),
    )(feat2d, op2d)


def _proj_body(x_ref, y_ref, z_ref, e_ref, k_ref, pix_ref):
    bf = jnp.bfloat16
    x = x_ref[0, 0]
    y = y_ref[0, 0]
    z = z_ref[0, 0]
    ones = jnp.ones_like(x)
    pts = jnp.stack([x, y, z, ones], axis=0)
    Eb = e_ref[0, 0].astype(bf)
    cam = lax.dot_general(Eb, pts.astype(bf), (((1,), (0,)), ((), ())),
                          preferred_element_type=jnp.float32)
    cz = cam[2]
    zs = jnp.maximum(cz, 1e-6)
    u0 = cam[0] / zs
    v0 = cam[1] / zs
    st = jnp.stack([u0, v0, jnp.ones_like(u0)], axis=0)
    Kb = k_ref[0, 0].astype(bf)
    uvh = lax.dot_general(Kb, st.astype(bf), (((1,), (0,)), ((), ())),
                          preferred_element_type=jnp.float32)
    xf = jnp.round(uvh[0])
    yf = jnp.round(uvh[1])
    idx = lax.iota(jnp.int32, NPAD)
    valid = (cz > 0) & (xf >= 0) & (xf < W) & (yf >= 0) & (yf < H) & (idx < N)
    pix_ref[...] = jnp.where(valid, (yf * W + xf).astype(jnp.int32), HW)


def _project(xp, yp, zp, E, K):
    return pl.pallas_call(
        _proj_body,
        grid=(B * V,),
        in_specs=[
            pl.BlockSpec((1, 1, NPAD), lambda i: (i // V, 0, 0)),
            pl.BlockSpec((1, 1, NPAD), lambda i: (i // V, 0, 0)),
            pl.BlockSpec((1, 1, NPAD), lambda i: (i // V, 0, 0)),
            pl.BlockSpec((1, 1, 4, 4), lambda i: (i // V, i % V, 0, 0)),
            pl.BlockSpec((1, 1, 3, 3), lambda i: (i // V, i % V, 0, 0)),
        ],
        out_specs=pl.BlockSpec((NPAD,), lambda i: (i,)),
        out_shape=jax.ShapeDtypeStruct((B * V * NPAD,), jnp.int32),
    )(xp, yp, zp, E, K)


_mesh = plsc.VectorSubcoreMesh(core_axis_name="c", subcore_axis_name="s")


@functools.partial(
    pl.kernel,
    out_type=[jax.ShapeDtypeStruct((B * V * 3 * HW,), jnp.float32),
              jax.ShapeDtypeStruct((B * V * HW,), jnp.float32),
              jax.ShapeDtypeStruct((B * V * HW,), jnp.int32)],
    mesh=_mesh,
    compiler_params=pltpu.CompilerParams(needs_layout_passes=False),
    scratch_types=[
        pltpu.VMEM((NPAD,), jnp.int32),
        pltpu.VMEM((2, CCHUNK), jnp.int32),
        pltpu.VMEM((2, CCHUNK), jnp.float32),
        pltpu.VMEM((CCHUNK,), jnp.float32),
        pltpu.VMEM((48,), jnp.int32),
        pltpu.SemaphoreType.DMA,
        pltpu.SemaphoreType.DMA,
        pltpu.SemaphoreType.DMA,
        pltpu.SemaphoreType.DMA,
        pltpu.SemaphoreType.DMA,
        pltpu.SemaphoreType.DMA,
    ],
)
def _sc_render(pix_hbm, tab_hbm, rgb_out, alpha_out, win_out,
               arena, winring, planering, zerobuf, chunkflag,
               sem_p0, sem_p1, sem_w0, sem_w1, sem_o0, sem_o1):
    cid = lax.axis_index("c")
    sid = lax.axis_index("s")
    wid = sid * 2 + cid
    view = wid >> 2
    q = wid & 3
    b = view >> 2
    lo = q * QPIX
    lane = lax.iota(jnp.int32, 16)
    zeros16f = jnp.zeros((16,), jnp.float32)
    zeros16i = jnp.zeros((16,), jnp.int32)
    ones16i = jnp.ones((16,), jnp.int32)
    neg16i = jnp.full((16,), -1, jnp.int32)

    def _initw(i, _):
        for u in range(4):
            arena[pl.ds(i * 64 + u * 16, 16)] = neg16i
        return 0
    lax.fori_loop(0, QPIX // 64, _initw, 0)

    def _initz(i, _):
        zerobuf[pl.ds(i * 16, 16)] = zeros16f
        return 0
    lax.fori_loop(0, CCHUNK // 16, _initz, 0)
    chunkflag[pl.ds(0, 16)] = zeros16i
    chunkflag[pl.ds(16, 16)] = zeros16i
    chunkflag[pl.ds(32, 16)] = zeros16i

    psems = [sem_p0, sem_p1]
    pbase = view * NPAD
    cur = pltpu.async_copy(pix_hbm.at[pl.ds(pbase, PCHUNK)],
                           arena.at[pl.ds(PIXOFF, PCHUNK)], psems[0])
    for c in range(NPC):
        nxt = None
        if c + 1 < NPC:
            nxt = pltpu.async_copy(
                pix_hbm.at[pl.ds(pbase + (c + 1) * PCHUNK, PCHUNK)],
                arena.at[pl.ds(PIXOFF + ((c + 1) % 2) * PCHUNK, PCHUNK)],
                psems[(c + 1) % 2])
        cur.wait()
        rbase = PIXOFF + (c % 2) * PCHUNK

        def _scan(g, _, c=c, rbase=rbase):
            for u in range(4):
                off = g * 64 + u * 16
                pixv = arena[pl.ds(rbase + off, 16)]
                inr = (pixv >= lo) & (pixv < lo + QPIX)
                loc = jnp.where(inr, pixv - lo, 0)
                idxv = (c * PCHUNK) + off + lane
                plsc.store_scatter(arena, [loc], idxv, mask=inr)
                plsc.store_scatter(chunkflag, [loc >> 11], ones16i, mask=inr)
            return 0
        lax.fori_loop(0, PCHUNK // 64, _scan, 0)
        cur = nxt

    wbase = view * HW + lo
    pltpu.async_copy(arena.at[pl.ds(0, QPIX)],
                     win_out.at[pl.ds(wbase, QPIX)], sem_p0).wait()

    wsems = [sem_w0, sem_w1]
    osems = [sem_o0, sem_o1]

    for ch in range(4):
        pltpu.async_copy(tab_hbm.at[pl.ds((ch * B + b) * NPAD, NPAD)],
                         arena.at[pl.ds(0, NPAD)], sem_p1).wait()
        if ch == 0:
            out_ref = alpha_out
            obase = view * HW + lo
        else:
            out_ref = rgb_out
            obase = (view * 3 + (ch - 1)) * HW + lo

        def _chunk(pc, p, i, out_ref=out_ref, obase=obase):
            flag = chunkflag[pl.ds(pc, 16)][0]
            dst = out_ref.at[pl.ds(obase + pc * CCHUNK, CCHUNK)]

            @pl.when(i >= 1)
            def _():
                pltpu.make_async_copy(planering.at[p], dst, osems[p]).wait()

            @pl.when(flag > 0)
            def _():
                pltpu.async_copy(
                    win_out.at[pl.ds(wbase + pc * CCHUNK, CCHUNK)],
                    winring.at[p], wsems[p]).wait()

                def _fill(g, _):
                    wv = winring[p, pl.ds(g * 16, 16)]
                    m = wv >= 0
                    vi = plsc.load_gather(arena, [jnp.maximum(wv, 0)], mask=m)
                    val = plsc.bitcast(vi, jnp.float32)
                    planering[p, pl.ds(g * 16, 16)] = jnp.where(m, val, 0.0)
                    return 0
                lax.fori_loop(0, CCHUNK // 16, _fill, 0)
                pltpu.async_copy(planering.at[p], dst, osems[p])

            @pl.when(flag == 0)
            def _():
                pltpu.async_copy(zerobuf, dst, osems[p])

        def _pair(i, _):
            _chunk(i * 2, 0, i)
            _chunk(i * 2 + 1, 1, i)
            return 0
        lax.fori_loop(0, NCC // 2, _pair, 0)

        for pcl, p in ((NCC - 2, 0), (NCC - 1, 1)):
            pltpu.make_async_copy(
                planering.at[p],
                out_ref.at[pl.ds(obase + pcl * CCHUNK, CCHUNK)],
                osems[p]).wait()


def kernel(gaussian_xyz, gaussian_opacity, gaussian_color_feat, intrinsics, extrinsics, image_size):
    featp = jnp.pad(gaussian_color_feat, ((0, 0), (0, NPAD - N), (0, 0)))
    opacp = jnp.pad(gaussian_opacity, ((0, 0), (0, NPAD - N), (0, 0)))
    tab_i = _records(featp.reshape(B * NPAD, 16),
                     opacp.reshape(B * NPAD, 1))

    xyzp = jnp.pad(gaussian_xyz, ((0, 0), (0, NPAD - N), (0, 0)))
    xp = xyzp[:, :, 0].reshape(B, 1, NPAD)
    yp = xyzp[:, :, 1].reshape(B, 1, NPAD)
    zp = xyzp[:, :, 2].reshape(B, 1, NPAD)
    pix = _project(xp, yp, zp, extrinsics, intrinsics)

    rgb, alpha, _ = _sc_render(pix, tab_i)
    return (rgb.reshape(B, V, 3, H, W), alpha.reshape(B, V, 1, H, W))


# speedup vs baseline: 1.9109x; 1.9109x over previous
import functools

import jax
import jax.numpy as jnp
from jax import lax
from jax.experimental import pallas as pl
from jax.experimental.pallas import tpu as pltpu, tpu_sc as plsc

B, V, N, H, W = 2, 4, 100000, 512, 512
HW = H * W
NPAD = 100352
PCHUNK = 12544
NPC = NPAD // PCHUNK
QPIX = HW // 4
CCHUNK = 2048
NCC = QPIX // CCHUNK
PIXOFF = QPIX


def _records_body(feat_ref, op_ref, out_ref):
    f = feat_ref[...]
    av = jnp.clip(op_ref[..., 0], 0.0, 1.0)
    r = jax.nn.sigmoid(f[:, 0])
    g = jax.nn.sigmoid(f[:, 1])
    b = jax.nn.sigmoid(f[:, 2])
    out_ref[...] = jnp.stack([av, r * av, g * av, b * av], axis=0)


def _records(feat2d, op2d):
    rows = feat2d.shape[0]
    blk = 7168
    return pl.pallas_call(
        _records_body,
        grid=(rows // blk,),
        in_specs=[
            pl.BlockSpec((blk, 16), lambda i: (i, 0)),
            pl.BlockSpec((blk, 1), lambda i: (i, 0)),
        ],
        out_specs=pl.BlockSpec((4, blk), lambda i: (0, i)),
        out_shape=jax.ShapeDtypeStruct((4, rows), jnp.float32),
    )(feat2d, op2d)


def _proj_body(x_ref, y_ref, z_ref, e_ref, k_ref, pix_ref):
    bf = jnp.bfloat16
    x = x_ref[0, 0]
    y = y_ref[0, 0]
    z = z_ref[0, 0]
    ones = jnp.ones_like(x)
    pts = jnp.stack([x, y, z, ones], axis=0)
    Eb = e_ref[0, 0].astype(bf)
    cam = lax.dot_general(Eb, pts.astype(bf), (((1,), (0,)), ((), ())),
                          preferred_element_type=jnp.float32)
    cz = cam[2]
    zs = jnp.maximum(cz, 1e-6)
    u0 = cam[0] / zs
    v0 = cam[1] / zs
    st = jnp.stack([u0, v0, jnp.ones_like(u0)], axis=0)
    Kb = k_ref[0, 0].astype(bf)
    uvh = lax.dot_general(Kb, st.astype(bf), (((1,), (0,)), ((), ())),
                          preferred_element_type=jnp.float32)
    xf = jnp.round(uvh[0])
    yf = jnp.round(uvh[1])
    idx = lax.iota(jnp.int32, NPAD)
    valid = (cz > 0) & (xf >= 0) & (xf < W) & (yf >= 0) & (yf < H) & (idx < N)
    pix_ref[0, 0] = jnp.where(valid, (yf * W + xf).astype(jnp.int32), HW)


def _project(xp, yp, zp, E, K):
    return pl.pallas_call(
        _proj_body,
        grid=(B * V,),
        in_specs=[
            pl.BlockSpec((1, 1, NPAD), lambda i: (i // V, 0, 0)),
            pl.BlockSpec((1, 1, NPAD), lambda i: (i // V, 0, 0)),
            pl.BlockSpec((1, 1, NPAD), lambda i: (i // V, 0, 0)),
            pl.BlockSpec((1, 1, 4, 4), lambda i: (i // V, i % V, 0, 0)),
            pl.BlockSpec((1, 1, 3, 3), lambda i: (i // V, i % V, 0, 0)),
        ],
        out_specs=pl.BlockSpec((1, 1, NPAD), lambda i: (i, 0, 0)),
        out_shape=jax.ShapeDtypeStruct((B * V, 1, NPAD), jnp.int32),
    )(xp, yp, zp, E, K)


_mesh = plsc.VectorSubcoreMesh(core_axis_name="c", subcore_axis_name="s")


@functools.partial(
    pl.kernel,
    out_type=[jax.ShapeDtypeStruct((B * V * 3 * HW,), jnp.float32),
              jax.ShapeDtypeStruct((B * V * HW,), jnp.float32),
              jax.ShapeDtypeStruct((B * V * HW,), jnp.int32)],
    mesh=_mesh,
    compiler_params=pltpu.CompilerParams(needs_layout_passes=False),
    scratch_types=[
        pltpu.VMEM((NPAD,), jnp.int32),
        pltpu.VMEM((2, CCHUNK), jnp.int32),
        pltpu.VMEM((2, CCHUNK), jnp.float32),
        pltpu.VMEM((CCHUNK,), jnp.float32),
        pltpu.VMEM((48,), jnp.int32),
        pltpu.SemaphoreType.DMA,
        pltpu.SemaphoreType.DMA,
        pltpu.SemaphoreType.DMA,
        pltpu.SemaphoreType.DMA,
        pltpu.SemaphoreType.DMA,
        pltpu.SemaphoreType.DMA,
    ],
)
def _sc_render(pix_hbm, tab_hbm, rgb_out, alpha_out, win_out,
               arena, winring, planering, zerobuf, chunkflag,
               sem_p0, sem_p1, sem_w0, sem_w1, sem_o0, sem_o1):
    cid = lax.axis_index("c")
    sid = lax.axis_index("s")
    wid = sid * 2 + cid
    view = wid >> 2
    q = wid & 3
    b = view >> 2
    lo = q * QPIX
    lane = lax.iota(jnp.int32, 16)
    zeros16f = jnp.zeros((16,), jnp.float32)
    zeros16i = jnp.zeros((16,), jnp.int32)
    ones16i = jnp.ones((16,), jnp.int32)
    neg16i = jnp.full((16,), -1, jnp.int32)

    def _initw(i, _):
        for u in range(4):
            arena[pl.ds(i * 64 + u * 16, 16)] = neg16i
        return 0
    lax.fori_loop(0, QPIX // 64, _initw, 0)

    def _initz(i, _):
        zerobuf[pl.ds(i * 16, 16)] = zeros16f
        return 0
    lax.fori_loop(0, CCHUNK // 16, _initz, 0)
    chunkflag[pl.ds(0, 16)] = zeros16i
    chunkflag[pl.ds(16, 16)] = zeros16i
    chunkflag[pl.ds(32, 16)] = zeros16i

    psems = [sem_p0, sem_p1]
    pbase = view * NPAD
    cur = pltpu.async_copy(pix_hbm.at[pl.ds(pbase, PCHUNK)],
                           arena.at[pl.ds(PIXOFF, PCHUNK)], psems[0])
    for c in range(NPC):
        nxt = None
        if c + 1 < NPC:
            nxt = pltpu.async_copy(
                pix_hbm.at[pl.ds(pbase + (c + 1) * PCHUNK, PCHUNK)],
                arena.at[pl.ds(PIXOFF + ((c + 1) % 2) * PCHUNK, PCHUNK)],
                psems[(c + 1) % 2])
        cur.wait()
        rbase = PIXOFF + (c % 2) * PCHUNK

        def _scan(g, _, c=c, rbase=rbase):
            for u in range(4):
                off = g * 64 + u * 16
                pixv = arena[pl.ds(rbase + off, 16)]
                inr = (pixv >= lo) & (pixv < lo + QPIX)
                loc = jnp.where(inr, pixv - lo, 0)
                idxv = (c * PCHUNK) + off + lane
                plsc.store_scatter(arena, [loc], idxv, mask=inr)
                plsc.store_scatter(chunkflag, [loc >> 11], ones16i, mask=inr)
            return 0
        lax.fori_loop(0, PCHUNK // 64, _scan, 0)
        cur = nxt

    wbase = view * HW + lo
    pltpu.async_copy(arena.at[pl.ds(0, QPIX)],
                     win_out.at[pl.ds(wbase, QPIX)], sem_p0).wait()

    wsems = [sem_w0, sem_w1]
    osems = [sem_o0, sem_o1]

    for ch in range(4):
        pltpu.async_copy(tab_hbm.at[pl.ds((ch * B + b) * NPAD, NPAD)],
                         arena.at[pl.ds(0, NPAD)], sem_p1).wait()
        if ch == 0:
            out_ref = alpha_out
            obase = view * HW + lo
        else:
            out_ref = rgb_out
            obase = (view * 3 + (ch - 1)) * HW + lo

        def _chunk(pc, p, i, out_ref=out_ref, obase=obase):
            flag = chunkflag[pl.ds(pc, 16)][0]
            dst = out_ref.at[pl.ds(obase + pc * CCHUNK, CCHUNK)]

            @pl.when(i >= 1)
            def _():
                pltpu.make_async_copy(planering.at[p], dst, osems[p]).wait()

            @pl.when(flag > 0)
            def _():
                pltpu.async_copy(
                    win_out.at[pl.ds(wbase + pc * CCHUNK, CCHUNK)],
                    winring.at[p], wsems[p]).wait()

                def _fill(g, _):
                    wv = winring[p, pl.ds(g * 16, 16)]
                    m = wv >= 0
                    vi = plsc.load_gather(arena, [jnp.maximum(wv, 0)], mask=m)
                    val = plsc.bitcast(vi, jnp.float32)
                    planering[p, pl.ds(g * 16, 16)] = jnp.where(m, val, 0.0)
                    return 0
                lax.fori_loop(0, CCHUNK // 16, _fill, 0)
                pltpu.async_copy(planering.at[p], dst, osems[p])

            @pl.when(flag == 0)
            def _():
                pltpu.async_copy(zerobuf, dst, osems[p])

        def _pair(i, _):
            _chunk(i * 2, 0, i)
            _chunk(i * 2 + 1, 1, i)
            return 0
        lax.fori_loop(0, NCC // 2, _pair, 0)

        for pcl, p in ((NCC - 2, 0), (NCC - 1, 1)):
            pltpu.make_async_copy(
                planering.at[p],
                out_ref.at[pl.ds(obase + pcl * CCHUNK, CCHUNK)],
                osems[p]).wait()


def kernel(gaussian_xyz, gaussian_opacity, gaussian_color_feat, intrinsics, extrinsics, image_size):
    featp = jnp.pad(gaussian_color_feat, ((0, 0), (0, NPAD - N), (0, 0)))
    opacp = jnp.pad(gaussian_opacity, ((0, 0), (0, NPAD - N), (0, 0)))
    tab = _records(featp.reshape(B * NPAD, 16),
                   opacp.reshape(B * NPAD, 1))
    tab_i = lax.bitcast_convert_type(tab.reshape(4 * B * NPAD), jnp.int32)

    xyzp = jnp.pad(gaussian_xyz, ((0, 0), (0, NPAD - N), (0, 0)))
    xp = xyzp[:, :, 0].reshape(B, 1, NPAD)
    yp = xyzp[:, :, 1].reshape(B, 1, NPAD)
    zp = xyzp[:, :, 2].reshape(B, 1, NPAD)
    pix = _project(xp, yp, zp, extrinsics, intrinsics)

    rgb, alpha, _ = _sc_render(pix.reshape(B * V * NPAD), tab_i)
    return (rgb.reshape(B, V, 3, H, W), alpha.reshape(B, V, 1, H, W))
